# Initial kernel scaffold; baseline (speedup 1.0000x reference)
import functools

import jax
import jax.numpy as jnp
from jax import lax
from jax.experimental import pallas as pl
from jax.experimental.pallas import tpu as pltpu
from jax.experimental.pallas import tpu_sc as plsc

N = 10000
E = 320000
DF = 128
DH = 256
NCLS = 40

N_PAD = 10240
DAUG = 144
DZ = 48

NC = 2
NS = 16
NW = NC * NS
EB = 128
NBLK = 2560
E_PAD = NBLK * EB
KB = NBLK // NW
STRIPE = N_PAD // NS

_f32 = jnp.float32
_i32 = jnp.int32


def _mesh():
    return plsc.VectorSubcoreMesh(
        core_axis_name="c", subcore_axis_name="s", num_cores=NC, num_subcores=NS
    )


def _deg_call(srcb, dstb, ones16, z16):
    @functools.partial(
        pl.kernel,
        out_type=(
            jax.ShapeDtypeStruct((NC, N_PAD, 16), _f32),
            jax.ShapeDtypeStruct((NC, N_PAD, 16), _f32),
        ),
        mesh=_mesh(),
        scratch_types=[
            pltpu.VMEM((KB, EB), _i32),
            pltpu.VMEM((KB, EB), _i32),
            pltpu.VMEM((EB, 16), _f32),
            pltpu.VMEM_SHARED((N_PAD, 16), _f32),
            pltpu.VMEM_SHARED((N_PAD, 16), _f32),
        ],
    )
    def deg_kernel(srcb_h, dstb_h, ones_h, z_h, dego, degi,
                   sidx, didx, ones_v, acco, acci):
        c = lax.axis_index("c")
        s = lax.axis_index("s")
        off = (c * NS + s) * KB
        r0 = s * STRIPE
        pltpu.sync_copy(srcb_h.at[pl.ds(off, KB)], sidx)
        pltpu.sync_copy(dstb_h.at[pl.ds(off, KB)], didx)
        pltpu.sync_copy(ones_h, ones_v)
        pltpu.sync_copy(z_h.at[pl.ds(r0, STRIPE)], acco.at[pl.ds(r0, STRIPE)])
        pltpu.sync_copy(z_h.at[pl.ds(r0, STRIPE)], acci.at[pl.ds(r0, STRIPE)])
        plsc.subcore_barrier()

        def body(j, carry):
            pltpu.sync_copy(ones_v, acco.at[sidx.at[j]], add=True)
            pltpu.sync_copy(ones_v, acci.at[didx.at[j]], add=True)
            return carry

        lax.fori_loop(0, KB, body, 0)
        plsc.subcore_barrier()
        pltpu.sync_copy(acco.at[pl.ds(r0, STRIPE)], dego.at[c, pl.ds(r0, STRIPE)])
        pltpu.sync_copy(acci.at[pl.ds(r0, STRIPE)], degi.at[c, pl.ds(r0, STRIPE)])

    return deg_kernel(srcb, dstb, ones16, z16)


def _segsum_call(rows, srcb, dstb, zD, D):
    @functools.partial(
        pl.kernel,
        out_type=jax.ShapeDtypeStruct((NC, N_PAD, D), _f32),
        mesh=_mesh(),
        scratch_types=[
            pltpu.VMEM((KB, EB), _i32),
            pltpu.VMEM((KB, EB), _i32),
            pltpu.VMEM((EB, D), _f32),
            pltpu.VMEM_SHARED((N_PAD, D), _f32),
            pltpu.SemaphoreType.DMA,
        ],
    )
    def seg_kernel(rows_h, srcb_h, dstb_h, z_h, out, sidx, didx, buf, acc, sem):
        c = lax.axis_index("c")
        s = lax.axis_index("s")
        off = (c * NS + s) * KB
        r0 = s * STRIPE
        pltpu.sync_copy(srcb_h.at[pl.ds(off, KB)], sidx)
        pltpu.sync_copy(dstb_h.at[pl.ds(off, KB)], didx)
        pltpu.sync_copy(z_h.at[pl.ds(r0, STRIPE)], acc.at[pl.ds(r0, STRIPE)])
        plsc.subcore_barrier()

        def body(j, carry):
            pltpu.async_copy(rows_h.at[sidx.at[j]], buf, sem).wait()
            pltpu.sync_copy(buf, acc.at[didx.at[j]], add=True)
            return carry

        lax.fori_loop(0, KB, body, 0)
        plsc.subcore_barrier()
        pltpu.sync_copy(acc.at[pl.ds(r0, STRIPE)], out.at[c, pl.ds(r0, STRIPE)])

    return seg_kernel(rows, srcb, dstb, zD)


def _tc1_call(xp, dego):
    R, G = 1024, N_PAD // 1024

    def body(x_ref, dg_ref, out_ref):
        d = dg_ref[0] + dg_ref[1]
        a = lax.rsqrt(d[:, 0:1] + 1.0)
        out_ref[...] = jnp.concatenate(
            [x_ref[...] * a, a, jnp.zeros((R, DAUG - DF - 1), _f32)], axis=1
        )

    return pl.pallas_call(
        body,
        grid=(G,),
        in_specs=[
            pl.BlockSpec((R, DF), lambda i: (i, 0)),
            pl.BlockSpec((NC, R, 16), lambda i: (0, i, 0)),
        ],
        out_specs=pl.BlockSpec((R, DAUG), lambda i: (i, 0)),
        out_shape=jax.ShapeDtypeStruct((N_PAD, DAUG), _f32),
    )(xp, dego)


def _tc2_call(agg, dego, degi, W1, b1r, W2p, b2p):
    R, G = 1024, N_PAD // 1024

    def body(agg_ref, dgo_ref, dgi_ref, w1_ref, b1_ref, w2_ref, b2_ref, out_ref):
        aggs = agg_ref[0] + agg_ref[1]
        dgo = dgo_ref[0] + dgo_ref[1]
        dgi = dgi_ref[0] + dgi_ref[1]
        a = lax.rsqrt(dgo[:, 0:1] + 1.0)
        b = lax.rsqrt(dgi[:, 0:1] + 1.0)
        aggx = aggs[:, :DF] * b
        sa = aggs[:, DF:DF + 1] * b
        h1 = jnp.dot(aggx, w1_ref[...], preferred_element_type=_f32)
        h1 = jnp.maximum(h1 + sa * b1_ref[...], 0.0)
        h2 = jnp.dot(h1, w2_ref[...], preferred_element_type=_f32) + b2_ref[...]
        out_ref[...] = a * h2

    return pl.pallas_call(
        body,
        grid=(G,),
        in_specs=[
            pl.BlockSpec((NC, R, DAUG), lambda i: (0, i, 0)),
            pl.BlockSpec((NC, R, 16), lambda i: (0, i, 0)),
            pl.BlockSpec((NC, R, 16), lambda i: (0, i, 0)),
            pl.BlockSpec((DF, DH), lambda i: (0, 0)),
            pl.BlockSpec((1, DH), lambda i: (0, 0)),
            pl.BlockSpec((DH, DZ), lambda i: (0, 0)),
            pl.BlockSpec((1, DZ), lambda i: (0, 0)),
        ],
        out_specs=pl.BlockSpec((R, DZ), lambda i: (i, 0)),
        out_shape=jax.ShapeDtypeStruct((N_PAD, DZ), _f32),
    )(agg, dego, degi, W1, b1r, W2p, b2p)


def _tc3_call(agg2, degi):
    R, G = 1000, 10

    def body(agg_ref, dgi_ref, out_ref):
        sm = agg_ref[0] + agg_ref[1]
        dgi = dgi_ref[0] + dgi_ref[1]
        b = lax.rsqrt(dgi[:, 0:1] + 1.0)
        logits = sm[:, :NCLS] * b
        m = jnp.max(logits, axis=1, keepdims=True)
        ex = jnp.exp(logits - m)
        lse = jnp.log(jnp.sum(ex, axis=1, keepdims=True))
        out_ref[...] = logits - m - lse

    return pl.pallas_call(
        body,
        grid=(G,),
        in_specs=[
            pl.BlockSpec((NC, R, DZ), lambda i: (0, i, 0)),
            pl.BlockSpec((NC, R, 16), lambda i: (0, i, 0)),
        ],
        out_specs=pl.BlockSpec((R, NCLS), lambda i: (i, 0)),
        out_shape=jax.ShapeDtypeStruct((N, NCLS), _f32),
    )(agg2, degi)


def kernel(x, edge_index, W1, b1, W2, b2):
    src = edge_index[0].astype(_i32)
    dst = edge_index[1].astype(_i32)
    padn = E_PAD - E
    srcb = jnp.concatenate([src, jnp.full((padn,), N, _i32)]).reshape(NBLK, EB)
    dstb = jnp.concatenate([dst, jnp.full((padn,), N, _i32)]).reshape(NBLK, EB)
    xp = jnp.pad(x, ((0, N_PAD - N), (0, 0)))

    z16 = jnp.zeros((N_PAD, 16), _f32)
    z144 = jnp.zeros((N_PAD, DAUG), _f32)
    z48 = jnp.zeros((N_PAD, DZ), _f32)
    ones16 = jnp.zeros((EB, 16), _f32).at[:, 0].set(1.0)

    dego, degi = _deg_call(srcb, dstb, ones16, z16)
    xaug = _tc1_call(xp, dego)
    agg = _segsum_call(xaug, srcb, dstb, z144, DAUG)
    W2p = jnp.pad(W2, ((0, 0), (0, DZ - NCLS)))
    b2p = jnp.pad(b2, (0, DZ - NCLS)).reshape(1, DZ)
    Z = _tc2_call(agg, dego, degi, W1, b1.reshape(1, DH), W2p, b2p)
    agg2 = _segsum_call(Z, srcb, dstb, z48, DZ)
    return _tc3_call(agg2, degi)


# speedup vs baseline: 8.5329x; 8.5329x over previous
import functools

import jax
import jax.numpy as jnp
from jax import lax
from jax.experimental import pallas as pl
from jax.experimental.pallas import tpu as pltpu
from jax.experimental.pallas import tpu_sc as plsc

N = 10000
E = 320000
DF = 128
DH = 256
NCLS = 40

N_PAD = 10240
DAUG = 144
DZ = 48

NC = 2
NS = 16
NW = NC * NS
EB = 128
NBLK = 2560
E_PAD = NBLK * EB
KB = NBLK // NW
STRIPE = N_PAD // NS

_f32 = jnp.float32
_i32 = jnp.int32


def _mesh():
    return plsc.VectorSubcoreMesh(
        core_axis_name="c", subcore_axis_name="s", num_cores=NC, num_subcores=NS
    )


def _deg_call(srcb, dstb, ones16, z16):
    @functools.partial(
        pl.kernel,
        out_type=(
            jax.ShapeDtypeStruct((NC, N_PAD, 16), _f32),
            jax.ShapeDtypeStruct((NC, N_PAD, 16), _f32),
        ),
        mesh=_mesh(),
        compiler_params=pltpu.CompilerParams(use_tc_tiling_on_sc=False),
        scratch_types=[
            pltpu.VMEM((KB, EB), _i32),
            pltpu.VMEM((KB, EB), _i32),
            pltpu.VMEM((EB, 16), _f32),
            pltpu.VMEM_SHARED((N_PAD, 16), _f32),
            pltpu.VMEM_SHARED((N_PAD, 16), _f32),
        ],
    )
    def deg_kernel(srcb_h, dstb_h, ones_h, z_h, dego, degi,
                   sidx, didx, ones_v, acco, acci):
        c = lax.axis_index("c")
        s = lax.axis_index("s")
        off = (c * NS + s) * KB
        r0 = s * STRIPE
        pltpu.sync_copy(srcb_h.at[pl.ds(off, KB)], sidx)
        pltpu.sync_copy(dstb_h.at[pl.ds(off, KB)], didx)
        pltpu.sync_copy(ones_h, ones_v)
        pltpu.sync_copy(z_h.at[pl.ds(r0, STRIPE)], acco.at[pl.ds(r0, STRIPE)])
        pltpu.sync_copy(z_h.at[pl.ds(r0, STRIPE)], acci.at[pl.ds(r0, STRIPE)])
        plsc.subcore_barrier()

        def body(j, carry):
            pltpu.sync_copy(ones_v, acco.at[sidx.at[j]], add=True)
            pltpu.sync_copy(ones_v, acci.at[didx.at[j]], add=True)
            return carry

        lax.fori_loop(0, KB, body, 0)
        plsc.subcore_barrier()
        pltpu.sync_copy(acco.at[pl.ds(r0, STRIPE)], dego.at[c, pl.ds(r0, STRIPE)])
        pltpu.sync_copy(acci.at[pl.ds(r0, STRIPE)], degi.at[c, pl.ds(r0, STRIPE)])

    return deg_kernel(srcb, dstb, ones16, z16)


def _segsum_call(rows, srcb, dstb, zD, D):
    @functools.partial(
        pl.kernel,
        out_type=jax.ShapeDtypeStruct((NC, N_PAD, D), _f32),
        mesh=_mesh(),
        compiler_params=pltpu.CompilerParams(use_tc_tiling_on_sc=False),
        scratch_types=[
            pltpu.VMEM((KB, EB), _i32),
            pltpu.VMEM((KB, EB), _i32),
            pltpu.VMEM((EB, D), _f32),
            pltpu.VMEM_SHARED((N_PAD, D), _f32),
            pltpu.SemaphoreType.DMA,
        ],
    )
    def seg_kernel(rows_h, srcb_h, dstb_h, z_h, out, sidx, didx, buf, acc, sem):
        c = lax.axis_index("c")
        s = lax.axis_index("s")
        off = (c * NS + s) * KB
        r0 = s * STRIPE
        pltpu.sync_copy(srcb_h.at[pl.ds(off, KB)], sidx)
        pltpu.sync_copy(dstb_h.at[pl.ds(off, KB)], didx)
        pltpu.sync_copy(z_h.at[pl.ds(r0, STRIPE)], acc.at[pl.ds(r0, STRIPE)])
        plsc.subcore_barrier()

        def body(j, carry):
            pltpu.async_copy(rows_h.at[sidx.at[j]], buf, sem).wait()
            pltpu.sync_copy(buf, acc.at[didx.at[j]], add=True)
            return carry

        lax.fori_loop(0, KB, body, 0)
        plsc.subcore_barrier()
        pltpu.sync_copy(acc.at[pl.ds(r0, STRIPE)], out.at[c, pl.ds(r0, STRIPE)])

    return seg_kernel(rows, srcb, dstb, zD)


def _tc1_call(xp, dego):
    R, G = 1024, N_PAD // 1024

    def body(x_ref, dg_ref, out_ref):
        d = dg_ref[0] + dg_ref[1]
        a = lax.rsqrt(d[:, 0:1] + 1.0)
        out_ref[...] = jnp.concatenate(
            [x_ref[...] * a, a, jnp.zeros((R, DAUG - DF - 1), _f32)], axis=1
        )

    return pl.pallas_call(
        body,
        grid=(G,),
        in_specs=[
            pl.BlockSpec((R, DF), lambda i: (i, 0)),
            pl.BlockSpec((NC, R, 16), lambda i: (0, i, 0)),
        ],
        out_specs=pl.BlockSpec((R, DAUG), lambda i: (i, 0)),
        out_shape=jax.ShapeDtypeStruct((N_PAD, DAUG), _f32),
    )(xp, dego)


def _tc2_call(agg, dego, degi, W1, b1r, W2p, b2p):
    R, G = 1024, N_PAD // 1024

    def body(agg_ref, dgo_ref, dgi_ref, w1_ref, b1_ref, w2_ref, b2_ref, out_ref):
        aggs = agg_ref[0] + agg_ref[1]
        dgo = dgo_ref[0] + dgo_ref[1]
        dgi = dgi_ref[0] + dgi_ref[1]
        a = lax.rsqrt(dgo[:, 0:1] + 1.0)
        b = lax.rsqrt(dgi[:, 0:1] + 1.0)
        aggx = aggs[:, :DF] * b
        sa = aggs[:, DF:DF + 1] * b
        h1 = jnp.dot(aggx, w1_ref[...], preferred_element_type=_f32)
        h1 = jnp.maximum(h1 + sa * b1_ref[...], 0.0)
        h2 = jnp.dot(h1, w2_ref[...], preferred_element_type=_f32) + b2_ref[...]
        out_ref[...] = a * h2

    return pl.pallas_call(
        body,
        grid=(G,),
        in_specs=[
            pl.BlockSpec((NC, R, DAUG), lambda i: (0, i, 0)),
            pl.BlockSpec((NC, R, 16), lambda i: (0, i, 0)),
            pl.BlockSpec((NC, R, 16), lambda i: (0, i, 0)),
            pl.BlockSpec((DF, DH), lambda i: (0, 0)),
            pl.BlockSpec((1, DH), lambda i: (0, 0)),
            pl.BlockSpec((DH, DZ), lambda i: (0, 0)),
            pl.BlockSpec((1, DZ), lambda i: (0, 0)),
        ],
        out_specs=pl.BlockSpec((R, DZ), lambda i: (i, 0)),
        out_shape=jax.ShapeDtypeStruct((N_PAD, DZ), _f32),
    )(agg, dego, degi, W1, b1r, W2p, b2p)


def _tc3_call(agg2, degi):
    R, G = 1000, 10

    def body(agg_ref, dgi_ref, out_ref):
        sm = agg_ref[0] + agg_ref[1]
        dgi = dgi_ref[0] + dgi_ref[1]
        b = lax.rsqrt(dgi[:, 0:1] + 1.0)
        logits = sm[:, :NCLS] * b
        m = jnp.max(logits, axis=1, keepdims=True)
        ex = jnp.exp(logits - m)
        lse = jnp.log(jnp.sum(ex, axis=1, keepdims=True))
        out_ref[...] = logits - m - lse

    return pl.pallas_call(
        body,
        grid=(G,),
        in_specs=[
            pl.BlockSpec((NC, R, DZ), lambda i: (0, i, 0)),
            pl.BlockSpec((NC, R, 16), lambda i: (0, i, 0)),
        ],
        out_specs=pl.BlockSpec((R, NCLS), lambda i: (i, 0)),
        out_shape=jax.ShapeDtypeStruct((N, NCLS), _f32),
    )(agg2, degi)


def kernel(x, edge_index, W1, b1, W2, b2):
    src = edge_index[0].astype(_i32)
    dst = edge_index[1].astype(_i32)
    padn = E_PAD - E
    srcb = jnp.concatenate([src, jnp.full((padn,), N, _i32)]).reshape(NBLK, EB)
    dstb = jnp.concatenate([dst, jnp.full((padn,), N, _i32)]).reshape(NBLK, EB)
    xp = jnp.pad(x, ((0, N_PAD - N), (0, 0)))

    z16 = jnp.zeros((N_PAD, 16), _f32)
    z144 = jnp.zeros((N_PAD, DAUG), _f32)
    z48 = jnp.zeros((N_PAD, DZ), _f32)
    ones16 = jnp.zeros((EB, 16), _f32).at[:, 0].set(1.0)

    dego, degi = _deg_call(srcb, dstb, ones16, z16)
    xaug = _tc1_call(xp, dego)
    agg = _segsum_call(xaug, srcb, dstb, z144, DAUG)
    W2p = jnp.pad(W2, ((0, 0), (0, DZ - NCLS)))
    b2p = jnp.pad(b2, (0, DZ - NCLS)).reshape(1, DZ)
    Z = _tc2_call(agg, dego, degi, W1, b1.reshape(1, DH), W2p, b2p)
    agg2 = _segsum_call(Z, srcb, dstb, z48, DZ)
    return _tc3_call(agg2, degi)


# speedup vs baseline: 25.1576x; 2.9483x over previous
import functools

import jax
import jax.numpy as jnp
from jax import lax
from jax.experimental import pallas as pl
from jax.experimental.pallas import tpu as pltpu
from jax.experimental.pallas import tpu_sc as plsc

N = 10000
E = 320000
DF = 128
DH = 256
NCLS = 40

N_PAD = 10240
DAUG = 144
DZ = 48

NC = 2
NS = 16
NW = NC * NS
EB = 128
NBLK = E // EB
KB_LO = NBLK // NW
KB_XTRA = NBLK % NW
KB_MAX = KB_LO + 1
NBLK_PAD = NW * KB_MAX
STRIPE = N_PAD // NS
NBUF = 4

_f32 = jnp.float32
_i32 = jnp.int32


def _mesh():
    return plsc.VectorSubcoreMesh(
        core_axis_name="c", subcore_axis_name="s", num_cores=NC, num_subcores=NS
    )


def _deg_call(srcb, dstb, ones16, z16):
    @functools.partial(
        pl.kernel,
        out_type=(
            jax.ShapeDtypeStruct((NC, N_PAD, 16), _f32),
            jax.ShapeDtypeStruct((NC, N_PAD, 16), _f32),
        ),
        mesh=_mesh(),
        compiler_params=pltpu.CompilerParams(use_tc_tiling_on_sc=False),
        scratch_types=[
            pltpu.VMEM((KB_MAX, EB), _i32),
            pltpu.VMEM((KB_MAX, EB), _i32),
            pltpu.VMEM((EB, 16), _f32),
            pltpu.VMEM_SHARED((N_PAD, 16), _f32),
            pltpu.VMEM_SHARED((N_PAD, 16), _f32),
        ],
    )
    def deg_kernel(srcb_h, dstb_h, ones_h, z_h, dego, degi,
                   sidx, didx, ones_v, acco, acci):
        c = lax.axis_index("c")
        s = lax.axis_index("s")
        w = c * NS + s
        off = w * KB_LO + jnp.minimum(w, KB_XTRA)
        cnt = KB_LO + (w < KB_XTRA).astype(_i32)
        r0 = s * STRIPE
        pltpu.sync_copy(srcb_h.at[pl.ds(off, KB_MAX)], sidx)
        pltpu.sync_copy(dstb_h.at[pl.ds(off, KB_MAX)], didx)
        pltpu.sync_copy(ones_h, ones_v)
        pltpu.sync_copy(z_h.at[pl.ds(r0, STRIPE)], acco.at[pl.ds(r0, STRIPE)])
        pltpu.sync_copy(z_h.at[pl.ds(r0, STRIPE)], acci.at[pl.ds(r0, STRIPE)])
        plsc.subcore_barrier()

        def body(j, carry):
            pltpu.sync_copy(ones_v, acco.at[sidx.at[j]], add=True)
            pltpu.sync_copy(ones_v, acci.at[didx.at[j]], add=True)
            return carry

        lax.fori_loop(0, cnt, body, 0)
        plsc.subcore_barrier()
        pltpu.sync_copy(acco.at[pl.ds(r0, STRIPE)], dego.at[c, pl.ds(r0, STRIPE)])
        pltpu.sync_copy(acci.at[pl.ds(r0, STRIPE)], degi.at[c, pl.ds(r0, STRIPE)])

    return deg_kernel(srcb, dstb, ones16, z16)


def _segsum_call(rows, srcb, dstb, zD, D):
    @functools.partial(
        pl.kernel,
        out_type=jax.ShapeDtypeStruct((NC, N_PAD, D), _f32),
        mesh=_mesh(),
        compiler_params=pltpu.CompilerParams(use_tc_tiling_on_sc=False),
        scratch_types=[
            pltpu.VMEM((KB_MAX, EB), _i32),
            pltpu.VMEM((KB_MAX, EB), _i32),
            [pltpu.VMEM((EB, D), _f32)] * NBUF,
            pltpu.VMEM_SHARED((N_PAD, D), _f32),
            [pltpu.SemaphoreType.DMA] * NBUF,
        ],
    )
    def seg_kernel(rows_h, srcb_h, dstb_h, z_h, out, sidx, didx, bufs, acc, sems):
        c = lax.axis_index("c")
        s = lax.axis_index("s")
        w = c * NS + s
        off = w * KB_LO + jnp.minimum(w, KB_XTRA)
        cnt = KB_LO + (w < KB_XTRA).astype(_i32)
        r0 = s * STRIPE
        pltpu.sync_copy(srcb_h.at[pl.ds(off, KB_MAX)], sidx)
        pltpu.sync_copy(dstb_h.at[pl.ds(off, KB_MAX)], didx)
        pltpu.sync_copy(z_h.at[pl.ds(r0, STRIPE)], acc.at[pl.ds(r0, STRIPE)])
        plsc.subcore_barrier()

        for b in range(NBUF):
            pltpu.async_copy(rows_h.at[sidx.at[b]], bufs[b], sems[b])

        def body(g, carry):
            for b in range(NBUF):
                j = g * NBUF + b

                @pl.when(j < cnt)
                def _():
                    pltpu.make_async_copy(
                        rows_h.at[sidx.at[j]], bufs[b], sems[b]
                    ).wait()
                    pltpu.sync_copy(bufs[b], acc.at[didx.at[j]], add=True)

                    @pl.when(j + NBUF < cnt)
                    def _():
                        pltpu.async_copy(
                            rows_h.at[sidx.at[j + NBUF]], bufs[b], sems[b]
                        )
            return carry

        lax.fori_loop(0, (KB_MAX + NBUF - 1) // NBUF, body, 0)
        plsc.subcore_barrier()
        pltpu.sync_copy(acc.at[pl.ds(r0, STRIPE)], out.at[c, pl.ds(r0, STRIPE)])

    return seg_kernel(rows, srcb, dstb, zD)


DHALF = DAUG // 2
CNT_LO = NBLK // NS
CNT_X = NBLK % NS
CNT_MAX = CNT_LO + 1
NBUF2 = 2


def _segsum_split_call(rows_pair, srcb, dstb, zH):
    @functools.partial(
        pl.kernel,
        out_type=jax.ShapeDtypeStruct((NC, N_PAD, DHALF), _f32),
        mesh=_mesh(),
        compiler_params=pltpu.CompilerParams(use_tc_tiling_on_sc=False),
        scratch_types=[
            pltpu.VMEM((CNT_MAX, EB), _i32),
            pltpu.VMEM((CNT_MAX, EB), _i32),
            [pltpu.VMEM((EB, DHALF), _f32)] * NBUF2,
            pltpu.VMEM_SHARED((N_PAD, DHALF), _f32),
            [pltpu.SemaphoreType.DMA] * NBUF2,
        ],
    )
    def seg_kernel(rows_h, srcb_h, dstb_h, z_h, out, sidx, didx, bufs, acc, sems):
        c = lax.axis_index("c")
        s = lax.axis_index("s")
        off = s * CNT_LO + jnp.minimum(s, CNT_X)
        cnt = CNT_LO + (s < CNT_X).astype(_i32)
        r0 = s * STRIPE
        pltpu.sync_copy(srcb_h.at[pl.ds(off, CNT_MAX)], sidx)
        pltpu.sync_copy(dstb_h.at[pl.ds(off, CNT_MAX)], didx)
        pltpu.sync_copy(z_h.at[pl.ds(r0, STRIPE)], acc.at[pl.ds(r0, STRIPE)])
        plsc.subcore_barrier()

        rows_c = rows_h.at[c]
        for b in range(NBUF2):
            pltpu.async_copy(rows_c.at[sidx.at[b]], bufs[b], sems[b])

        def body(g, carry):
            for b in range(NBUF2):
                j = g * NBUF2 + b

                @pl.when(j < cnt)
                def _():
                    pltpu.make_async_copy(
                        rows_c.at[sidx.at[j]], bufs[b], sems[b]
                    ).wait()
                    pltpu.sync_copy(bufs[b], acc.at[didx.at[j]], add=True)

                    @pl.when(j + NBUF2 < cnt)
                    def _():
                        pltpu.async_copy(
                            rows_c.at[sidx.at[j + NBUF2]], bufs[b], sems[b]
                        )
            return carry

        lax.fori_loop(0, (CNT_MAX + NBUF2 - 1) // NBUF2, body, 0)
        plsc.subcore_barrier()
        pltpu.sync_copy(acc.at[pl.ds(r0, STRIPE)], out.at[c, pl.ds(r0, STRIPE)])

    return seg_kernel(rows_pair, srcb, dstb, zH)


def _tc1_call(xp, dego):
    R, G = 1024, N_PAD // 1024

    def body(x_ref, dg_ref, out_ref):
        d = dg_ref[0] + dg_ref[1]
        a = lax.rsqrt(d[:, 0:1] + 1.0)
        xs = x_ref[...] * a
        out_ref[0] = xs[:, :DHALF]
        out_ref[1] = jnp.concatenate(
            [xs[:, DHALF:], a, jnp.zeros((R, DAUG - DF - 1), _f32)], axis=1
        )

    return pl.pallas_call(
        body,
        grid=(G,),
        in_specs=[
            pl.BlockSpec((R, DF), lambda i: (i, 0)),
            pl.BlockSpec((NC, R, 16), lambda i: (0, i, 0)),
        ],
        out_specs=pl.BlockSpec((NC, R, DHALF), lambda i: (0, i, 0)),
        out_shape=jax.ShapeDtypeStruct((NC, N_PAD, DHALF), _f32),
    )(xp, dego)


def _tc2_call(agg, dego, degi, W1, b1r, W2p, b2p):
    R, G = 1024, N_PAD // 1024

    def body(agg_ref, dgo_ref, dgi_ref, w1_ref, b1_ref, w2_ref, b2_ref, out_ref):
        aggs = jnp.concatenate([agg_ref[0], agg_ref[1]], axis=1)
        dgo = dgo_ref[0] + dgo_ref[1]
        dgi = dgi_ref[0] + dgi_ref[1]
        a = lax.rsqrt(dgo[:, 0:1] + 1.0)
        b = lax.rsqrt(dgi[:, 0:1] + 1.0)
        aggx = aggs[:, :DF] * b
        sa = aggs[:, DF:DF + 1] * b
        h1 = jnp.dot(aggx, w1_ref[...], preferred_element_type=_f32)
        h1 = jnp.maximum(h1 + sa * b1_ref[...], 0.0)
        h2 = jnp.dot(h1, w2_ref[...], preferred_element_type=_f32) + b2_ref[...]
        out_ref[...] = a * h2

    return pl.pallas_call(
        body,
        grid=(G,),
        in_specs=[
            pl.BlockSpec((NC, R, DHALF), lambda i: (0, i, 0)),
            pl.BlockSpec((NC, R, 16), lambda i: (0, i, 0)),
            pl.BlockSpec((NC, R, 16), lambda i: (0, i, 0)),
            pl.BlockSpec((DF, DH), lambda i: (0, 0)),
            pl.BlockSpec((1, DH), lambda i: (0, 0)),
            pl.BlockSpec((DH, DZ), lambda i: (0, 0)),
            pl.BlockSpec((1, DZ), lambda i: (0, 0)),
        ],
        out_specs=pl.BlockSpec((R, DZ), lambda i: (i, 0)),
        out_shape=jax.ShapeDtypeStruct((N_PAD, DZ), _f32),
    )(agg, dego, degi, W1, b1r, W2p, b2p)


def _tc3_call(agg2, degi):
    R, G = 1000, 10

    def body(agg_ref, dgi_ref, out_ref):
        sm = agg_ref[0] + agg_ref[1]
        dgi = dgi_ref[0] + dgi_ref[1]
        b = lax.rsqrt(dgi[:, 0:1] + 1.0)
        logits = sm[:, :NCLS] * b
        m = jnp.max(logits, axis=1, keepdims=True)
        ex = jnp.exp(logits - m)
        lse = jnp.log(jnp.sum(ex, axis=1, keepdims=True))
        out_ref[...] = logits - m - lse

    return pl.pallas_call(
        body,
        grid=(G,),
        in_specs=[
            pl.BlockSpec((NC, R, DZ), lambda i: (0, i, 0)),
            pl.BlockSpec((NC, R, 16), lambda i: (0, i, 0)),
        ],
        out_specs=pl.BlockSpec((R, NCLS), lambda i: (i, 0)),
        out_shape=jax.ShapeDtypeStruct((N, NCLS), _f32),
    )(agg2, degi)


def kernel(x, edge_index, W1, b1, W2, b2):
    src = edge_index[0].astype(_i32)
    dst = edge_index[1].astype(_i32)
    padn = NBLK_PAD * EB - E
    srcb = jnp.concatenate([src, jnp.zeros((padn,), _i32)]).reshape(NBLK_PAD, EB)
    dstb = jnp.concatenate([dst, jnp.zeros((padn,), _i32)]).reshape(NBLK_PAD, EB)
    xp = jnp.pad(x, ((0, N_PAD - N), (0, 0)))

    z16 = jnp.zeros((N_PAD, 16), _f32)
    z72 = jnp.zeros((N_PAD, DHALF), _f32)
    z48 = jnp.zeros((N_PAD, DZ), _f32)
    ones16 = jnp.zeros((EB, 16), _f32).at[:, 0].set(1.0)

    dego, degi = _deg_call(srcb, dstb, ones16, z16)
    xaug = _tc1_call(xp, dego)
    agg = _segsum_split_call(xaug, srcb, dstb, z72)
    W2p = jnp.pad(W2, ((0, 0), (0, DZ - NCLS)))
    b2p = jnp.pad(b2, (0, DZ - NCLS)).reshape(1, DZ)
    Z = _tc2_call(agg, dego, degi, W1, b1.reshape(1, DH), W2p, b2p)
    agg2 = _segsum_call(Z, srcb, dstb, z48, DZ)
    return _tc3_call(agg2, degi)


# speedup vs baseline: 25.5976x; 1.0175x over previous
import functools

import jax
import jax.numpy as jnp
from jax import lax
from jax.experimental import pallas as pl
from jax.experimental.pallas import tpu as pltpu
from jax.experimental.pallas import tpu_sc as plsc

N = 10000
E = 320000
DF = 128
DH = 256
NCLS = 40

N_PAD = 10240
DAUG = 144
DHALF = DAUG // 2
DZ = 48

NC = 2
NS = 16
NW = NC * NS
EB = 128
NBLK = E // EB
KB_LO = NBLK // NW
KB_XTRA = NBLK % NW
KB_MAX = KB_LO + 1
CNT_LO = NBLK // NS
CNT_X = NBLK % NS
CNT_MAX = CNT_LO + 1
NBLK_PAD = NW * KB_MAX
STRIPE = N_PAD // NS

GD = 2
SD = 2
NB = GD + SD

_f32 = jnp.float32
_i32 = jnp.int32


def _mesh():
    return plsc.VectorSubcoreMesh(
        core_axis_name="c", subcore_axis_name="s", num_cores=NC, num_subcores=NS
    )


_sc_params = pltpu.CompilerParams(use_tc_tiling_on_sc=False)


def _deg_call(srcb, dstb, ones16, z16):
    @functools.partial(
        pl.kernel,
        out_type=(
            jax.ShapeDtypeStruct((NC, N_PAD, 16), _f32),
            jax.ShapeDtypeStruct((NC, N_PAD, 16), _f32),
        ),
        mesh=_mesh(),
        compiler_params=_sc_params,
        scratch_types=[
            pltpu.VMEM((KB_MAX, EB), _i32),
            pltpu.VMEM((KB_MAX, EB), _i32),
            pltpu.VMEM((EB, 16), _f32),
            pltpu.VMEM_SHARED((N_PAD, 16), _f32),
            pltpu.VMEM_SHARED((N_PAD, 16), _f32),
            [pltpu.SemaphoreType.DMA] * NB,
            [pltpu.SemaphoreType.DMA] * NB,
        ],
    )
    def deg_kernel(srcb_h, dstb_h, ones_h, z_h, dego, degi,
                   sidx, didx, ones_v, acco, acci, osems, isems):
        c = lax.axis_index("c")
        s = lax.axis_index("s")
        w = c * NS + s
        off = w * KB_LO + jnp.minimum(w, KB_XTRA)
        cnt = KB_LO + (w < KB_XTRA).astype(_i32)
        r0 = s * STRIPE
        pltpu.sync_copy(srcb_h.at[pl.ds(off, KB_MAX)], sidx)
        pltpu.sync_copy(dstb_h.at[pl.ds(off, KB_MAX)], didx)
        pltpu.sync_copy(ones_h, ones_v)
        pltpu.sync_copy(z_h.at[pl.ds(r0, STRIPE)], acco.at[pl.ds(r0, STRIPE)])
        pltpu.sync_copy(z_h.at[pl.ds(r0, STRIPE)], acci.at[pl.ds(r0, STRIPE)])
        plsc.subcore_barrier()

        def body(g, carry):
            for u in range(NB):
                j = g * NB + u

                @pl.when(j < cnt)
                def _():
                    @pl.when(j >= NB)
                    def _():
                        pltpu.make_async_copy(
                            ones_v, acco.at[sidx.at[0]], osems[u]
                        ).wait()
                        pltpu.make_async_copy(
                            ones_v, acci.at[didx.at[0]], isems[u]
                        ).wait()

                    pltpu.async_copy(
                        ones_v, acco.at[sidx.at[j]], osems[u], add=True
                    )
                    pltpu.async_copy(
                        ones_v, acci.at[didx.at[j]], isems[u], add=True
                    )
            return carry

        lax.fori_loop(0, (KB_MAX + NB - 1) // NB, body, 0)
        for u in range(NB):
            pltpu.make_async_copy(ones_v, acco.at[sidx.at[0]], osems[u]).wait()
            pltpu.make_async_copy(ones_v, acci.at[didx.at[0]], isems[u]).wait()
        plsc.subcore_barrier()
        pltpu.sync_copy(acco.at[pl.ds(r0, STRIPE)], dego.at[c, pl.ds(r0, STRIPE)])
        pltpu.sync_copy(acci.at[pl.ds(r0, STRIPE)], degi.at[c, pl.ds(r0, STRIPE)])

    return deg_kernel(srcb, dstb, ones16, z16)


def _segsum_call(rows, srcb, dstb, zD, D, col_split):
    CM = CNT_MAX if col_split else KB_MAX

    @functools.partial(
        pl.kernel,
        out_type=jax.ShapeDtypeStruct((NC, N_PAD, D), _f32),
        mesh=_mesh(),
        compiler_params=_sc_params,
        scratch_types=[
            pltpu.VMEM((CM, EB), _i32),
            pltpu.VMEM((CM, EB), _i32),
            [pltpu.VMEM((EB, D), _f32)] * NB,
            pltpu.VMEM_SHARED((N_PAD, D), _f32),
            [pltpu.SemaphoreType.DMA] * NB,
            [pltpu.SemaphoreType.DMA] * NB,
        ],
    )
    def seg_kernel(rows_h, srcb_h, dstb_h, z_h, out,
                   sidx, didx, bufs, acc, gsems, ssems):
        c = lax.axis_index("c")
        s = lax.axis_index("s")
        if col_split:
            off = s * CNT_LO + jnp.minimum(s, CNT_X)
            cnt = CNT_LO + (s < CNT_X).astype(_i32)
            rows_c = rows_h.at[c]
        else:
            w = c * NS + s
            off = w * KB_LO + jnp.minimum(w, KB_XTRA)
            cnt = KB_LO + (w < KB_XTRA).astype(_i32)
            rows_c = rows_h
        r0 = s * STRIPE
        pltpu.sync_copy(srcb_h.at[pl.ds(off, CM)], sidx)
        pltpu.sync_copy(dstb_h.at[pl.ds(off, CM)], didx)
        pltpu.sync_copy(z_h.at[pl.ds(r0, STRIPE)], acc.at[pl.ds(r0, STRIPE)])
        plsc.subcore_barrier()

        for b in range(GD):
            pltpu.async_copy(rows_c.at[sidx.at[b]], bufs[b], gsems[b])

        def body(g, carry):
            for u in range(NB):
                j = g * NB + u
                t = (u + GD) % NB

                @pl.when(j < cnt)
                def _():
                    pltpu.make_async_copy(
                        rows_c.at[sidx.at[j]], bufs[u], gsems[u]
                    ).wait()
                    pltpu.async_copy(
                        bufs[u], acc.at[didx.at[j]], ssems[u], add=True
                    )

                    @pl.when(j + GD < cnt)
                    def _():
                        @pl.when(j >= SD)
                        def _():
                            pltpu.make_async_copy(
                                bufs[t], acc.at[didx.at[0]], ssems[t]
                            ).wait()

                        pltpu.async_copy(
                            rows_c.at[sidx.at[j + GD]], bufs[t], gsems[t]
                        )
            return carry

        lax.fori_loop(0, (CM + NB - 1) // NB, body, 0)
        for b in range(NB):
            pltpu.make_async_copy(bufs[b], acc.at[didx.at[0]], ssems[b]).wait()
        plsc.subcore_barrier()
        pltpu.sync_copy(acc.at[pl.ds(r0, STRIPE)], out.at[c, pl.ds(r0, STRIPE)])

    return seg_kernel(rows, srcb, dstb, zD)


def _tc1_call(xp, dego):
    R, G = 1024, N_PAD // 1024

    def body(x_ref, dg_ref, out_ref):
        d = dg_ref[0] + dg_ref[1]
        a = lax.rsqrt(d[:, 0:1] + 1.0)
        xs = x_ref[...] * a
        out_ref[0] = xs[:, :DHALF]
        out_ref[1] = jnp.concatenate(
            [xs[:, DHALF:], a, jnp.zeros((R, DAUG - DF - 1), _f32)], axis=1
        )

    return pl.pallas_call(
        body,
        grid=(G,),
        in_specs=[
            pl.BlockSpec((R, DF), lambda i: (i, 0)),
            pl.BlockSpec((NC, R, 16), lambda i: (0, i, 0)),
        ],
        out_specs=pl.BlockSpec((NC, R, DHALF), lambda i: (0, i, 0)),
        out_shape=jax.ShapeDtypeStruct((NC, N_PAD, DHALF), _f32),
    )(xp, dego)


def _tc2_call(agg, dego, degi, W1, b1r, W2p, b2p):
    R, G = 1024, N_PAD // 1024

    def body(agg_ref, dgo_ref, dgi_ref, w1_ref, b1_ref, w2_ref, b2_ref, out_ref):
        aggs = jnp.concatenate([agg_ref[0], agg_ref[1]], axis=1)
        dgo = dgo_ref[0] + dgo_ref[1]
        dgi = dgi_ref[0] + dgi_ref[1]
        a = lax.rsqrt(dgo[:, 0:1] + 1.0)
        b = lax.rsqrt(dgi[:, 0:1] + 1.0)
        aggx = aggs[:, :DF] * b
        sa = aggs[:, DF:DF + 1] * b
        h1 = jnp.dot(aggx, w1_ref[...], preferred_element_type=_f32)
        h1 = jnp.maximum(h1 + sa * b1_ref[...], 0.0)
        h2 = jnp.dot(h1, w2_ref[...], preferred_element_type=_f32) + b2_ref[...]
        out_ref[...] = a * h2

    return pl.pallas_call(
        body,
        grid=(G,),
        in_specs=[
            pl.BlockSpec((NC, R, DHALF), lambda i: (0, i, 0)),
            pl.BlockSpec((NC, R, 16), lambda i: (0, i, 0)),
            pl.BlockSpec((NC, R, 16), lambda i: (0, i, 0)),
            pl.BlockSpec((DF, DH), lambda i: (0, 0)),
            pl.BlockSpec((1, DH), lambda i: (0, 0)),
            pl.BlockSpec((DH, DZ), lambda i: (0, 0)),
            pl.BlockSpec((1, DZ), lambda i: (0, 0)),
        ],
        out_specs=pl.BlockSpec((R, DZ), lambda i: (i, 0)),
        out_shape=jax.ShapeDtypeStruct((N_PAD, DZ), _f32),
    )(agg, dego, degi, W1, b1r, W2p, b2p)


def _tc3_call(agg2, degi):
    R, G = 1000, 10

    def body(agg_ref, dgi_ref, out_ref):
        sm = agg_ref[0] + agg_ref[1]
        dgi = dgi_ref[0] + dgi_ref[1]
        b = lax.rsqrt(dgi[:, 0:1] + 1.0)
        logits = sm[:, :NCLS] * b
        m = jnp.max(logits, axis=1, keepdims=True)
        ex = jnp.exp(logits - m)
        lse = jnp.log(jnp.sum(ex, axis=1, keepdims=True))
        out_ref[...] = logits - m - lse

    return pl.pallas_call(
        body,
        grid=(G,),
        in_specs=[
            pl.BlockSpec((NC, R, DZ), lambda i: (0, i, 0)),
            pl.BlockSpec((NC, R, 16), lambda i: (0, i, 0)),
        ],
        out_specs=pl.BlockSpec((R, NCLS), lambda i: (i, 0)),
        out_shape=jax.ShapeDtypeStruct((N, NCLS), _f32),
    )(agg2, degi)


def kernel(x, edge_index, W1, b1, W2, b2):
    src = edge_index[0].astype(_i32)
    dst = edge_index[1].astype(_i32)
    padn = NBLK_PAD * EB - E
    srcb = jnp.concatenate([src, jnp.zeros((padn,), _i32)]).reshape(NBLK_PAD, EB)
    dstb = jnp.concatenate([dst, jnp.zeros((padn,), _i32)]).reshape(NBLK_PAD, EB)
    xp = jnp.pad(x, ((0, N_PAD - N), (0, 0)))

    z16 = jnp.zeros((N_PAD, 16), _f32)
    z72 = jnp.zeros((N_PAD, DHALF), _f32)
    z48 = jnp.zeros((N_PAD, DZ), _f32)
    ones16 = jnp.zeros((EB, 16), _f32).at[:, 0].set(1.0)

    dego, degi = _deg_call(srcb, dstb, ones16, z16)
    xaug = _tc1_call(xp, dego)
    agg = _segsum_call(xaug, srcb, dstb, z72, DHALF, col_split=True)
    W2p = jnp.pad(W2, ((0, 0), (0, DZ - NCLS)))
    b2p = jnp.pad(b2, (0, DZ - NCLS)).reshape(1, DZ)
    Z = _tc2_call(agg, dego, degi, W1, b1.reshape(1, DH), W2p, b2p)
    agg2 = _segsum_call(Z, srcb, dstb, z48, DZ, col_split=False)
    return _tc3_call(agg2, degi)


# speedup vs baseline: 26.1314x; 1.0209x over previous
import functools

import jax
import jax.numpy as jnp
from jax import lax
from jax.experimental import pallas as pl
from jax.experimental.pallas import tpu as pltpu
from jax.experimental.pallas import tpu_sc as plsc

N = 10000
E = 320000
DF = 128
DH = 256
NCLS = 40

N_PAD = 10240
DAUG = 144
DHALF = DAUG // 2
DZ = 48

NC = 2
NS = 16
NW = NC * NS
EB = 128
NBLK = E // EB
KB_LO = NBLK // NW
KB_XTRA = NBLK % NW
KB_MAX = KB_LO + 1
CNT_LO = NBLK // NS
CNT_X = NBLK % NS
CNT_MAX = CNT_LO + 1
NBLK_PAD = NW * KB_MAX
STRIPE = N_PAD // NS

GD = 2
SD = 2
NB = GD + SD

_f32 = jnp.float32
_i32 = jnp.int32


def _mesh():
    return plsc.VectorSubcoreMesh(
        core_axis_name="c", subcore_axis_name="s", num_cores=NC, num_subcores=NS
    )


_sc_params = pltpu.CompilerParams(use_tc_tiling_on_sc=False)


def _deg_call(srcb, dstb, ones16, z16):
    @functools.partial(
        pl.kernel,
        out_type=(
            jax.ShapeDtypeStruct((NC, N_PAD, 1), _f32),
            jax.ShapeDtypeStruct((NC, N_PAD, 1), _f32),
        ),
        mesh=_mesh(),
        compiler_params=_sc_params,
        scratch_types=[
            pltpu.VMEM((KB_MAX, EB), _i32),
            pltpu.VMEM((KB_MAX, EB), _i32),
            pltpu.VMEM((EB, 1), _f32),
            pltpu.VMEM_SHARED((N_PAD, 1), _f32),
            pltpu.VMEM_SHARED((N_PAD, 1), _f32),
            [pltpu.SemaphoreType.DMA] * NB,
            [pltpu.SemaphoreType.DMA] * NB,
        ],
    )
    def deg_kernel(srcb_h, dstb_h, ones_h, z_h, dego, degi,
                   sidx, didx, ones_v, acco, acci, osems, isems):
        c = lax.axis_index("c")
        s = lax.axis_index("s")
        w = c * NS + s
        off = w * KB_LO + jnp.minimum(w, KB_XTRA)
        cnt = KB_LO + (w < KB_XTRA).astype(_i32)
        r0 = s * STRIPE
        pltpu.sync_copy(srcb_h.at[pl.ds(off, KB_MAX)], sidx)
        pltpu.sync_copy(dstb_h.at[pl.ds(off, KB_MAX)], didx)
        pltpu.sync_copy(ones_h, ones_v)
        pltpu.sync_copy(z_h.at[pl.ds(r0, STRIPE)], acco.at[pl.ds(r0, STRIPE)])
        pltpu.sync_copy(z_h.at[pl.ds(r0, STRIPE)], acci.at[pl.ds(r0, STRIPE)])
        plsc.subcore_barrier()

        def body(g, carry):
            for u in range(NB):
                j = g * NB + u

                @pl.when(j < cnt)
                def _():
                    @pl.when(j >= NB)
                    def _():
                        pltpu.make_async_copy(
                            ones_v, acco.at[sidx.at[0]], osems[u]
                        ).wait()
                        pltpu.make_async_copy(
                            ones_v, acci.at[didx.at[0]], isems[u]
                        ).wait()

                    pltpu.async_copy(
                        ones_v, acco.at[sidx.at[j]], osems[u], add=True
                    )
                    pltpu.async_copy(
                        ones_v, acci.at[didx.at[j]], isems[u], add=True
                    )
            return carry

        lax.fori_loop(0, (KB_MAX + NB - 1) // NB, body, 0)
        for u in range(NB):
            pltpu.make_async_copy(ones_v, acco.at[sidx.at[0]], osems[u]).wait()
            pltpu.make_async_copy(ones_v, acci.at[didx.at[0]], isems[u]).wait()
        plsc.subcore_barrier()
        pltpu.sync_copy(acco.at[pl.ds(r0, STRIPE)], dego.at[c, pl.ds(r0, STRIPE)])
        pltpu.sync_copy(acci.at[pl.ds(r0, STRIPE)], degi.at[c, pl.ds(r0, STRIPE)])

    return deg_kernel(srcb, dstb, ones16, z16)


def _segsum_call(rows, srcb, dstb, zD, D, col_split):
    CM = CNT_MAX if col_split else KB_MAX

    @functools.partial(
        pl.kernel,
        out_type=jax.ShapeDtypeStruct((NC, N_PAD, D), _f32),
        mesh=_mesh(),
        compiler_params=_sc_params,
        scratch_types=[
            pltpu.VMEM((CM, EB), _i32),
            pltpu.VMEM((CM, EB), _i32),
            [pltpu.VMEM((EB, D), _f32)] * NB,
            pltpu.VMEM_SHARED((N_PAD, D), _f32),
            [pltpu.SemaphoreType.DMA] * NB,
            [pltpu.SemaphoreType.DMA] * NB,
        ],
    )
    def seg_kernel(rows_h, srcb_h, dstb_h, z_h, out,
                   sidx, didx, bufs, acc, gsems, ssems):
        c = lax.axis_index("c")
        s = lax.axis_index("s")
        if col_split:
            off = s * CNT_LO + jnp.minimum(s, CNT_X)
            cnt = CNT_LO + (s < CNT_X).astype(_i32)
            rows_c = rows_h.at[c]
        else:
            w = c * NS + s
            off = w * KB_LO + jnp.minimum(w, KB_XTRA)
            cnt = KB_LO + (w < KB_XTRA).astype(_i32)
            rows_c = rows_h
        r0 = s * STRIPE
        pltpu.sync_copy(srcb_h.at[pl.ds(off, CM)], sidx)
        pltpu.sync_copy(dstb_h.at[pl.ds(off, CM)], didx)
        pltpu.sync_copy(z_h.at[pl.ds(r0, STRIPE)], acc.at[pl.ds(r0, STRIPE)])
        plsc.subcore_barrier()

        for b in range(GD):
            pltpu.async_copy(rows_c.at[sidx.at[b]], bufs[b], gsems[b])

        def body(g, carry):
            for u in range(NB):
                j = g * NB + u
                t = (u + GD) % NB

                @pl.when(j < cnt)
                def _():
                    pltpu.make_async_copy(
                        rows_c.at[sidx.at[j]], bufs[u], gsems[u]
                    ).wait()
                    pltpu.async_copy(
                        bufs[u], acc.at[didx.at[j]], ssems[u], add=True
                    )

                    @pl.when(j + GD < cnt)
                    def _():
                        @pl.when(j >= SD)
                        def _():
                            pltpu.make_async_copy(
                                bufs[t], acc.at[didx.at[0]], ssems[t]
                            ).wait()

                        pltpu.async_copy(
                            rows_c.at[sidx.at[j + GD]], bufs[t], gsems[t]
                        )
            return carry

        lax.fori_loop(0, (CM + NB - 1) // NB, body, 0)
        for b in range(NB):
            pltpu.make_async_copy(bufs[b], acc.at[didx.at[0]], ssems[b]).wait()
        plsc.subcore_barrier()
        pltpu.sync_copy(acc.at[pl.ds(r0, STRIPE)], out.at[c, pl.ds(r0, STRIPE)])

    return seg_kernel(rows, srcb, dstb, zD)


def _tc1_call(xp, dego):
    R, G = 1024, N_PAD // 1024

    def body(x_ref, dg_ref, out_ref):
        d = dg_ref[0] + dg_ref[1]
        a = lax.rsqrt(d[:, 0:1] + 1.0)
        xs = x_ref[...] * a
        out_ref[0] = xs[:, :DHALF]
        out_ref[1] = jnp.concatenate(
            [xs[:, DHALF:], a, jnp.zeros((R, DAUG - DF - 1), _f32)], axis=1
        )

    return pl.pallas_call(
        body,
        grid=(G,),
        in_specs=[
            pl.BlockSpec((R, DF), lambda i: (i, 0)),
            pl.BlockSpec((NC, R, 1), lambda i: (0, i, 0)),
        ],
        out_specs=pl.BlockSpec((NC, R, DHALF), lambda i: (0, i, 0)),
        out_shape=jax.ShapeDtypeStruct((NC, N_PAD, DHALF), _f32),
    )(xp, dego)


def _tc2_call(agg, dego, degi, W1, b1r, W2p, b2p):
    R, G = 1024, N_PAD // 1024

    def body(agg_ref, dgo_ref, dgi_ref, w1_ref, b1_ref, w2_ref, b2_ref, out_ref):
        aggs = jnp.concatenate([agg_ref[0], agg_ref[1]], axis=1)
        dgo = dgo_ref[0] + dgo_ref[1]
        dgi = dgi_ref[0] + dgi_ref[1]
        a = lax.rsqrt(dgo[:, 0:1] + 1.0)
        b = lax.rsqrt(dgi[:, 0:1] + 1.0)
        aggx = aggs[:, :DF] * b
        sa = aggs[:, DF:DF + 1] * b
        h1 = jnp.dot(aggx, w1_ref[...], preferred_element_type=_f32)
        h1 = jnp.maximum(h1 + sa * b1_ref[...], 0.0)
        h2 = jnp.dot(h1, w2_ref[...], preferred_element_type=_f32) + b2_ref[...]
        out_ref[...] = a * h2

    return pl.pallas_call(
        body,
        grid=(G,),
        in_specs=[
            pl.BlockSpec((NC, R, DHALF), lambda i: (0, i, 0)),
            pl.BlockSpec((NC, R, 1), lambda i: (0, i, 0)),
            pl.BlockSpec((NC, R, 1), lambda i: (0, i, 0)),
            pl.BlockSpec((DF, DH), lambda i: (0, 0)),
            pl.BlockSpec((1, DH), lambda i: (0, 0)),
            pl.BlockSpec((DH, DZ), lambda i: (0, 0)),
            pl.BlockSpec((1, DZ), lambda i: (0, 0)),
        ],
        out_specs=pl.BlockSpec((R, DZ), lambda i: (i, 0)),
        out_shape=jax.ShapeDtypeStruct((N_PAD, DZ), _f32),
    )(agg, dego, degi, W1, b1r, W2p, b2p)


def _tc3_call(agg2, degi):
    R, G = 1000, 10

    def body(agg_ref, dgi_ref, out_ref):
        sm = agg_ref[0] + agg_ref[1]
        dgi = dgi_ref[0] + dgi_ref[1]
        b = lax.rsqrt(dgi[:, 0:1] + 1.0)
        logits = sm[:, :NCLS] * b
        m = jnp.max(logits, axis=1, keepdims=True)
        ex = jnp.exp(logits - m)
        lse = jnp.log(jnp.sum(ex, axis=1, keepdims=True))
        out_ref[...] = logits - m - lse

    return pl.pallas_call(
        body,
        grid=(G,),
        in_specs=[
            pl.BlockSpec((NC, R, DZ), lambda i: (0, i, 0)),
            pl.BlockSpec((NC, R, 1), lambda i: (0, i, 0)),
        ],
        out_specs=pl.BlockSpec((R, NCLS), lambda i: (i, 0)),
        out_shape=jax.ShapeDtypeStruct((N, NCLS), _f32),
    )(agg2, degi)


def kernel(x, edge_index, W1, b1, W2, b2):
    src = edge_index[0].astype(_i32)
    dst = edge_index[1].astype(_i32)
    padn = NBLK_PAD * EB - E
    srcb = jnp.concatenate([src, jnp.zeros((padn,), _i32)]).reshape(NBLK_PAD, EB)
    dstb = jnp.concatenate([dst, jnp.zeros((padn,), _i32)]).reshape(NBLK_PAD, EB)
    xp = jnp.pad(x, ((0, N_PAD - N), (0, 0)))

    z1 = jnp.zeros((N_PAD, 1), _f32)
    z72 = jnp.zeros((N_PAD, DHALF), _f32)
    z48 = jnp.zeros((N_PAD, DZ), _f32)
    ones1 = jnp.ones((EB, 1), _f32)

    dego, degi = _deg_call(srcb, dstb, ones1, z1)
    xaug = _tc1_call(xp, dego)
    agg = _segsum_call(xaug, srcb, dstb, z72, DHALF, col_split=True)
    W2p = jnp.pad(W2, ((0, 0), (0, DZ - NCLS)))
    b2p = jnp.pad(b2, (0, DZ - NCLS)).reshape(1, DZ)
    Z = _tc2_call(agg, dego, degi, W1, b1.reshape(1, DH), W2p, b2p)
    agg2 = _segsum_call(Z, srcb, dstb, z48, DZ, col_split=False)
    return _tc3_call(agg2, degi)


# speedup vs baseline: 26.5683x; 1.0167x over previous
import functools

import jax
import jax.numpy as jnp
from jax import lax
from jax.experimental import pallas as pl
from jax.experimental.pallas import tpu as pltpu
from jax.experimental.pallas import tpu_sc as plsc

N = 10000
E = 320000
DF = 128
DH = 256
NCLS = 40

N_PAD = 10240
DAUG = 144
DHALF = DAUG // 2
DZ = 48

NC = 2
NS = 16
NW = NC * NS
EB = 128
NBLK = E // EB
KB_LO = NBLK // NW
KB_XTRA = NBLK % NW
KB_MAX = KB_LO + 1
CNT_LO = NBLK // NS
CNT_X = NBLK % NS
CNT_MAX = CNT_LO + 1
NBLK_PAD = NW * KB_MAX
STRIPE = N_PAD // NS

GD = 2
SD = 2
NB = GD + SD

_f32 = jnp.float32
_i32 = jnp.int32


def _mesh():
    return plsc.VectorSubcoreMesh(
        core_axis_name="c", subcore_axis_name="s", num_cores=NC, num_subcores=NS
    )


_sc_params = pltpu.CompilerParams(use_tc_tiling_on_sc=False)


def _deg_call(edges, ones16, z16):
    @functools.partial(
        pl.kernel,
        out_type=(
            jax.ShapeDtypeStruct((NC, N_PAD, 16), _f32),
            jax.ShapeDtypeStruct((NC, N_PAD, 16), _f32),
        ),
        mesh=_mesh(),
        compiler_params=_sc_params,
        scratch_types=[
            pltpu.VMEM((KB_MAX, EB), _i32),
            pltpu.VMEM((KB_MAX, EB), _i32),
            pltpu.VMEM((EB, 16), _f32),
            pltpu.VMEM_SHARED((N_PAD, 16), _f32),
            pltpu.VMEM_SHARED((N_PAD, 16), _f32),
            [pltpu.SemaphoreType.DMA] * NB,
            [pltpu.SemaphoreType.DMA] * NB,
        ],
    )
    def deg_kernel(edges_h, ones_h, z_h, dego, degi,
                   sidx, didx, ones_v, acco, acci, osems, isems):
        c = lax.axis_index("c")
        s = lax.axis_index("s")
        w = c * NS + s
        off = w * KB_LO + jnp.minimum(w, KB_XTRA)
        cnt = KB_LO + (w < KB_XTRA).astype(_i32)
        woff = jnp.minimum(off, NBLK - KB_MAX)
        dj = off - woff
        r0 = s * STRIPE
        pltpu.sync_copy(edges_h.at[0, pl.ds(woff, KB_MAX)], sidx)
        pltpu.sync_copy(edges_h.at[1, pl.ds(woff, KB_MAX)], didx)
        pltpu.sync_copy(ones_h, ones_v)
        pltpu.sync_copy(z_h.at[pl.ds(r0, STRIPE)], acco.at[pl.ds(r0, STRIPE)])
        pltpu.sync_copy(z_h.at[pl.ds(r0, STRIPE)], acci.at[pl.ds(r0, STRIPE)])
        plsc.subcore_barrier()

        def body(g, carry):
            for u in range(NB):
                j = g * NB + u

                @pl.when(j < cnt)
                def _():
                    @pl.when(j >= NB)
                    def _():
                        pltpu.make_async_copy(
                            ones_v, acco.at[sidx.at[0]], osems[u]
                        ).wait()
                        pltpu.make_async_copy(
                            ones_v, acci.at[didx.at[0]], isems[u]
                        ).wait()

                    pltpu.async_copy(
                        ones_v, acco.at[sidx.at[j + dj]], osems[u], add=True
                    )
                    pltpu.async_copy(
                        ones_v, acci.at[didx.at[j + dj]], isems[u], add=True
                    )
            return carry

        lax.fori_loop(0, (KB_MAX + NB - 1) // NB, body, 0)
        for u in range(NB):
            pltpu.make_async_copy(ones_v, acco.at[sidx.at[0]], osems[u]).wait()
            pltpu.make_async_copy(ones_v, acci.at[didx.at[0]], isems[u]).wait()
        plsc.subcore_barrier()
        pltpu.sync_copy(acco.at[pl.ds(r0, STRIPE)], dego.at[c, pl.ds(r0, STRIPE)])
        pltpu.sync_copy(acci.at[pl.ds(r0, STRIPE)], degi.at[c, pl.ds(r0, STRIPE)])

    return deg_kernel(edges, ones16, z16)


def _segsum_call(rows, edges, zD, D, col_split):
    CM = CNT_MAX if col_split else KB_MAX

    @functools.partial(
        pl.kernel,
        out_type=jax.ShapeDtypeStruct((NC, N_PAD, D), _f32),
        mesh=_mesh(),
        compiler_params=_sc_params,
        scratch_types=[
            pltpu.VMEM((CM, EB), _i32),
            pltpu.VMEM((CM, EB), _i32),
            [pltpu.VMEM((EB, D), _f32)] * NB,
            pltpu.VMEM_SHARED((N_PAD, D), _f32),
            [pltpu.SemaphoreType.DMA] * NB,
            [pltpu.SemaphoreType.DMA] * NB,
        ],
    )
    def seg_kernel(rows_h, edges_h, z_h, out,
                   sidx, didx, bufs, acc, gsems, ssems):
        c = lax.axis_index("c")
        s = lax.axis_index("s")
        if col_split:
            off = s * CNT_LO + jnp.minimum(s, CNT_X)
            cnt = CNT_LO + (s < CNT_X).astype(_i32)
            rows_c = rows_h.at[c]
        else:
            w = c * NS + s
            off = w * KB_LO + jnp.minimum(w, KB_XTRA)
            cnt = KB_LO + (w < KB_XTRA).astype(_i32)
            rows_c = rows_h
        woff = jnp.minimum(off, NBLK - CM)
        dj = off - woff
        r0 = s * STRIPE
        pltpu.sync_copy(edges_h.at[0, pl.ds(woff, CM)], sidx)
        pltpu.sync_copy(edges_h.at[1, pl.ds(woff, CM)], didx)
        pltpu.sync_copy(z_h.at[pl.ds(r0, STRIPE)], acc.at[pl.ds(r0, STRIPE)])
        plsc.subcore_barrier()

        for b in range(GD):
            pltpu.async_copy(rows_c.at[sidx.at[b + dj]], bufs[b], gsems[b])

        def body(g, carry):
            for u in range(NB):
                j = g * NB + u
                t = (u + GD) % NB

                @pl.when(j < cnt)
                def _():
                    pltpu.make_async_copy(
                        rows_c.at[sidx.at[j + dj]], bufs[u], gsems[u]
                    ).wait()
                    pltpu.async_copy(
                        bufs[u], acc.at[didx.at[j + dj]], ssems[u], add=True
                    )

                    @pl.when(j + GD < cnt)
                    def _():
                        @pl.when(j >= SD)
                        def _():
                            pltpu.make_async_copy(
                                bufs[t], acc.at[didx.at[0]], ssems[t]
                            ).wait()

                        pltpu.async_copy(
                            rows_c.at[sidx.at[j + GD + dj]], bufs[t], gsems[t]
                        )
            return carry

        lax.fori_loop(0, (CM + NB - 1) // NB, body, 0)
        for b in range(NB):
            pltpu.make_async_copy(bufs[b], acc.at[didx.at[0]], ssems[b]).wait()
        plsc.subcore_barrier()
        pltpu.sync_copy(acc.at[pl.ds(r0, STRIPE)], out.at[c, pl.ds(r0, STRIPE)])

    return seg_kernel(rows, edges, zD)


def _tc1_call(x, dego):
    R, G = 1000, 10

    def body(x_ref, dg_ref, out_ref):
        d = dg_ref[0] + dg_ref[1]
        a = lax.rsqrt(d[:, 0:1] + 1.0)
        xs = x_ref[...] * a
        out_ref[0] = xs[:, :DHALF]
        out_ref[1] = jnp.concatenate(
            [xs[:, DHALF:], a, jnp.zeros((R, DAUG - DF - 1), _f32)], axis=1
        )

    return pl.pallas_call(
        body,
        grid=(G,),
        in_specs=[
            pl.BlockSpec((R, DF), lambda i: (i, 0)),
            pl.BlockSpec((NC, R, 16), lambda i: (0, i, 0)),
        ],
        out_specs=pl.BlockSpec((NC, R, DHALF), lambda i: (0, i, 0)),
        out_shape=jax.ShapeDtypeStruct((NC, N, DHALF), _f32),
    )(x, dego)


def _tc2_call(agg, dego, degi, W1, b1r, W2p, b2p):
    R, G = 1000, 10

    def body(agg_ref, dgo_ref, dgi_ref, w1_ref, b1_ref, w2_ref, b2_ref, out_ref):
        aggs = jnp.concatenate([agg_ref[0], agg_ref[1]], axis=1)
        dgo = dgo_ref[0] + dgo_ref[1]
        dgi = dgi_ref[0] + dgi_ref[1]
        a = lax.rsqrt(dgo[:, 0:1] + 1.0)
        b = lax.rsqrt(dgi[:, 0:1] + 1.0)
        aggx = aggs[:, :DF] * b
        sa = aggs[:, DF:DF + 1] * b
        h1 = jnp.dot(aggx, w1_ref[...], preferred_element_type=_f32)
        h1 = jnp.maximum(h1 + sa * b1_ref[...], 0.0)
        h2 = jnp.dot(h1, w2_ref[...], preferred_element_type=_f32) + b2_ref[...]
        out_ref[...] = a * h2

    return pl.pallas_call(
        body,
        grid=(G,),
        in_specs=[
            pl.BlockSpec((NC, R, DHALF), lambda i: (0, i, 0)),
            pl.BlockSpec((NC, R, 16), lambda i: (0, i, 0)),
            pl.BlockSpec((NC, R, 16), lambda i: (0, i, 0)),
            pl.BlockSpec((DF, DH), lambda i: (0, 0)),
            pl.BlockSpec((1, DH), lambda i: (0, 0)),
            pl.BlockSpec((DH, DZ), lambda i: (0, 0)),
            pl.BlockSpec((1, DZ), lambda i: (0, 0)),
        ],
        out_specs=pl.BlockSpec((R, DZ), lambda i: (i, 0)),
        out_shape=jax.ShapeDtypeStruct((N, DZ), _f32),
    )(agg, dego, degi, W1, b1r, W2p, b2p)


def _tc3_call(agg2, degi):
    R, G = 1000, 10

    def body(agg_ref, dgi_ref, out_ref):
        sm = agg_ref[0] + agg_ref[1]
        dgi = dgi_ref[0] + dgi_ref[1]
        b = lax.rsqrt(dgi[:, 0:1] + 1.0)
        logits = sm[:, :NCLS] * b
        m = jnp.max(logits, axis=1, keepdims=True)
        ex = jnp.exp(logits - m)
        lse = jnp.log(jnp.sum(ex, axis=1, keepdims=True))
        out_ref[...] = logits - m - lse

    return pl.pallas_call(
        body,
        grid=(G,),
        in_specs=[
            pl.BlockSpec((NC, R, DZ), lambda i: (0, i, 0)),
            pl.BlockSpec((NC, R, 16), lambda i: (0, i, 0)),
        ],
        out_specs=pl.BlockSpec((R, NCLS), lambda i: (i, 0)),
        out_shape=jax.ShapeDtypeStruct((N, NCLS), _f32),
    )(agg2, degi)


def kernel(x, edge_index, W1, b1, W2, b2):
    edges = edge_index.astype(_i32).reshape(2, NBLK, EB)

    z16 = jnp.zeros((N_PAD, 16), _f32)
    z72 = jnp.zeros((N_PAD, DHALF), _f32)
    z48 = jnp.zeros((N_PAD, DZ), _f32)
    ones16 = jnp.zeros((EB, 16), _f32).at[:, 0].set(1.0)

    dego, degi = _deg_call(edges, ones16, z16)
    xaug = _tc1_call(x, dego)
    agg = _segsum_call(xaug, edges, z72, DHALF, col_split=True)
    W2p = jnp.pad(W2, ((0, 0), (0, DZ - NCLS)))
    b2p = jnp.pad(b2, (0, DZ - NCLS)).reshape(1, DZ)
    Z = _tc2_call(agg, dego, degi, W1, b1.reshape(1, DH), W2p, b2p)
    agg2 = _segsum_call(Z, edges, z48, DZ, col_split=False)
    return _tc3_call(agg2, degi)
